# baseline (speedup 1.0000x reference)
import functools

import jax
import jax.numpy as jnp
from jax import lax
from jax.experimental import pallas as pl
from jax.experimental.pallas import tpu as pltpu
from jax.experimental.pallas import tpu_sc as plsc

_FEAT = 512
_NBUCKET = 100
_BATCH = 16384
_START_SMOOTH = 1

_NC, _NS, _L = 2, 16, 16
_NW = _NC * _NS
_RPW = _BATCH // _NW
_CH = 16
_CW = _CH * _FEAT
_NCHUNK = _RPW // _CH
_NBUF = 3
_TABW = _NBUCKET * _FEAT


def _tables_body(ep_ref, rv_ref, sv_ref, rm_ref, sm_ref, so_ref):
    live = ep_ref[0, 0] >= _START_SMOOTH
    s = jnp.sqrt(jnp.clip(sv_ref[...] / rv_ref[...], 0.1, 10.0))
    s = jnp.where(live, s, 1.0)
    o = jnp.where(live, sm_ref[...] - rm_ref[...] * s, 0.0)
    so_ref[0, ...] = s
    so_ref[1, ...] = o


def _make_tables(ep, rv, sv, rm, sm):
    return pl.pallas_call(
        _tables_body,
        in_specs=[
            pl.BlockSpec(memory_space=pltpu.SMEM),
            pl.BlockSpec(memory_space=pltpu.VMEM),
            pl.BlockSpec(memory_space=pltpu.VMEM),
            pl.BlockSpec(memory_space=pltpu.VMEM),
            pl.BlockSpec(memory_space=pltpu.VMEM),
        ],
        out_shape=jax.ShapeDtypeStruct((2, _NBUCKET, _FEAT), jnp.float32),
    )(ep, rv, sv, rm, sm)


@functools.partial(
    pl.kernel,
    out_type=jax.ShapeDtypeStruct((_BATCH * _FEAT,), jnp.float32),
    mesh=plsc.VectorSubcoreMesh(core_axis_name="c", subcore_axis_name="s"),
    compiler_params=pltpu.CompilerParams(needs_layout_passes=False),
    scratch_types=[
        pltpu.VMEM((_RPW,), jnp.int32),
        pltpu.VMEM((_TABW,), jnp.float32),
        pltpu.VMEM((_TABW,), jnp.float32),
        pltpu.VMEM((_NBUF * _CW,), jnp.float32),
        pltpu.SemaphoreType.DMA,
        pltpu.SemaphoreType.DMA,
        pltpu.SemaphoreType.DMA,
        pltpu.SemaphoreType.DMA,
        pltpu.SemaphoreType.DMA,
        pltpu.SemaphoreType.DMA,
        pltpu.SemaphoreType.DMA,
        pltpu.SemaphoreType.DMA,
    ],
)
def _sc_apply(feat_hbm, lab_hbm, so_hbm, out_hbm,
              idx_v, tabs_v, tabo_v, f_v,
              sts, sto, si0, si1, si2, so0, so1, so2):
    sin = (si0, si1, si2)
    sout = (so0, so1, so2)
    wid = lax.axis_index("s") * _NC + lax.axis_index("c")
    base = wid * _RPW
    pltpu.async_copy(so_hbm.at[pl.ds(0, _TABW)], tabs_v, sts)
    pltpu.async_copy(so_hbm.at[pl.ds(_TABW, _TABW)], tabo_v, sto)
    pltpu.sync_copy(lab_hbm.at[pl.ds(base, _RPW)], idx_v)

    def issue_in(ci, b):
        pltpu.async_copy(
            feat_hbm.at[pl.ds((base + ci * _CH) * _FEAT, _CW)],
            f_v.at[pl.ds(b * _CW, _CW)], sin[b])

    def wait_in(b):
        pltpu.make_async_copy(
            feat_hbm.at[pl.ds(0, _CW)],
            f_v.at[pl.ds(b * _CW, _CW)], sin[b]).wait()

    def issue_out(ci, b):
        pltpu.async_copy(
            f_v.at[pl.ds(b * _CW, _CW)],
            out_hbm.at[pl.ds((base + ci * _CH) * _FEAT, _CW)], sout[b])

    def wait_out(b):
        pltpu.make_async_copy(
            f_v.at[pl.ds(b * _CW, _CW)],
            out_hbm.at[pl.ds(0, _CW)], sout[b]).wait()

    issue_in(0, 0)
    issue_in(1, 1)
    pltpu.make_async_copy(so_hbm.at[pl.ds(0, _TABW)], tabs_v, sts).wait()
    pltpu.make_async_copy(so_hbm.at[pl.ds(0, _TABW)], tabo_v, sto).wait()

    iot = lax.iota(jnp.int32, 16)

    for ci in range(_NCHUNK):
        b = ci % _NBUF
        wait_in(b)

        lv = idx_v[pl.ds(ci * _CH, _CH)]
        lvs = lv * jnp.int32(_FEAT)
        fbase = iot * jnp.int32(_FEAT) + jnp.int32(b * _CW)

        def col_body(c, c2, lvs=lvs, fbase=fbase):
            fidx = fbase + c
            sidx = lvs + c
            fx = plsc.load_gather(f_v, [fidx])
            sc = plsc.load_gather(tabs_v, [sidx])
            of = plsc.load_gather(tabo_v, [sidx])
            plsc.store_scatter(f_v, [fidx], fx * sc + of)
            return c2

        lax.fori_loop(0, _FEAT, col_body, 0, unroll=4)
        issue_out(ci, b)

        if ci + 2 < _NCHUNK:
            bq = (ci + 2) % _NBUF
            if ci >= 1:
                wait_out(bq)
            issue_in(ci + 2, bq)

    for k in (_NCHUNK - 3, _NCHUNK - 2, _NCHUNK - 1):
        wait_out(k % _NBUF)


def kernel(features, labels, epoch,
           running_mean_last_epoch, running_var_last_epoch,
           smoothed_mean_last_epoch, smoothed_var_last_epoch):
    lab = jnp.clip(labels.reshape(-1).astype(jnp.int32), 0, _NBUCKET - 1)
    ep = jnp.asarray(epoch, jnp.int32).reshape(1, 1)
    so = _make_tables(
        ep, running_var_last_epoch, smoothed_var_last_epoch,
        running_mean_last_epoch, smoothed_mean_last_epoch)
    out = _sc_apply(features.reshape(-1), lab, so.reshape(-1))
    return out.reshape(_BATCH, _FEAT)


# speedup vs baseline: 1.6377x; 1.6377x over previous
import functools

import jax
import jax.numpy as jnp
from jax import lax
from jax.experimental import pallas as pl
from jax.experimental.pallas import tpu as pltpu
from jax.experimental.pallas import tpu_sc as plsc

_FEAT = 512
_NBUCKET = 100
_BATCH = 16384
_START_SMOOTH = 1

_NC, _NS, _L = 2, 16, 16
_NW = _NC * _NS
_RPW = _BATCH // _NW
_CH = 16
_CW = _CH * _FEAT
_NCHUNK = _RPW // _CH
_NBUF = 3
_TABW = _NBUCKET * _FEAT


def _tables_body(ep_ref, rv_ref, sv_ref, rm_ref, sm_ref, so_ref):
    live = ep_ref[0, 0] >= _START_SMOOTH
    s = jnp.sqrt(jnp.clip(sv_ref[...] / rv_ref[...], 0.1, 10.0))
    s = jnp.where(live, s, 1.0)
    o = jnp.where(live, sm_ref[...] - rm_ref[...] * s, 0.0)
    so_ref[0, ...] = s
    so_ref[1, ...] = o


def _make_tables(ep, rv, sv, rm, sm):
    return pl.pallas_call(
        _tables_body,
        in_specs=[
            pl.BlockSpec(memory_space=pltpu.SMEM),
            pl.BlockSpec(memory_space=pltpu.VMEM),
            pl.BlockSpec(memory_space=pltpu.VMEM),
            pl.BlockSpec(memory_space=pltpu.VMEM),
            pl.BlockSpec(memory_space=pltpu.VMEM),
        ],
        out_shape=jax.ShapeDtypeStruct((2, _NBUCKET, _FEAT), jnp.float32),
    )(ep, rv, sv, rm, sm)


@functools.partial(
    pl.kernel,
    out_type=jax.ShapeDtypeStruct((_BATCH * _FEAT,), jnp.float32),
    mesh=plsc.VectorSubcoreMesh(core_axis_name="c", subcore_axis_name="s"),
    compiler_params=pltpu.CompilerParams(needs_layout_passes=False),
    scratch_types=[
        pltpu.VMEM((_RPW,), jnp.int32),
        pltpu.VMEM((_TABW,), jnp.float32),
        pltpu.VMEM((_TABW,), jnp.float32),
        pltpu.VMEM((_NBUF * _CW,), jnp.float32),
        pltpu.SemaphoreType.DMA,
        pltpu.SemaphoreType.DMA,
        pltpu.SemaphoreType.DMA,
        pltpu.SemaphoreType.DMA,
        pltpu.SemaphoreType.DMA,
        pltpu.SemaphoreType.DMA,
        pltpu.SemaphoreType.DMA,
        pltpu.SemaphoreType.DMA,
    ],
)
def _sc_apply(feat_hbm, lab_hbm, so_hbm, out_hbm,
              idx_v, tabs_v, tabo_v, f_v,
              sts, sto, si0, si1, si2, so0, so1, so2):
    sin = (si0, si1, si2)
    sout = (so0, so1, so2)
    wid = lax.axis_index("s") * _NC + lax.axis_index("c")
    base = wid * _RPW
    pltpu.async_copy(so_hbm.at[pl.ds(0, _TABW)], tabs_v, sts)
    pltpu.async_copy(so_hbm.at[pl.ds(_TABW, _TABW)], tabo_v, sto)
    pltpu.sync_copy(lab_hbm.at[pl.ds(base, _RPW)], idx_v)

    def issue_in(ci, b):
        pltpu.async_copy(
            feat_hbm.at[pl.ds((base + ci * _CH) * _FEAT, _CW)],
            f_v.at[pl.ds(b * _CW, _CW)], sin[b])

    def wait_in(b):
        pltpu.make_async_copy(
            feat_hbm.at[pl.ds(0, _CW)],
            f_v.at[pl.ds(b * _CW, _CW)], sin[b]).wait()

    def issue_out(ci, b):
        pltpu.async_copy(
            f_v.at[pl.ds(b * _CW, _CW)],
            out_hbm.at[pl.ds((base + ci * _CH) * _FEAT, _CW)], sout[b])

    def wait_out(b):
        pltpu.make_async_copy(
            f_v.at[pl.ds(b * _CW, _CW)],
            out_hbm.at[pl.ds(0, _CW)], sout[b]).wait()

    issue_in(0, 0)
    issue_in(1, 1)
    pltpu.make_async_copy(so_hbm.at[pl.ds(0, _TABW)], tabs_v, sts).wait()
    pltpu.make_async_copy(so_hbm.at[pl.ds(0, _TABW)], tabo_v, sto).wait()

    iot = lax.iota(jnp.int32, 16)

    for ci in range(_NCHUNK):
        b = ci % _NBUF
        wait_in(b)

        lv = idx_v[pl.ds(ci * _CH, _CH)]
        lvs = lv * jnp.int32(_FEAT)
        fbase = iot * jnp.int32(_FEAT) + jnp.int32(b * _CW)

        @plsc.parallel_loop(0, _FEAT, unroll=8)
        def _(c, lvs=lvs, fbase=fbase):
            fidx = fbase + c
            sidx = lvs + c
            fx = plsc.load_gather(f_v, [fidx])
            sc = plsc.load_gather(tabs_v, [sidx])
            of = plsc.load_gather(tabo_v, [sidx])
            plsc.store_scatter(f_v, [fidx], fx * sc + of)
        issue_out(ci, b)

        if ci + 2 < _NCHUNK:
            bq = (ci + 2) % _NBUF
            if ci >= 1:
                wait_out(bq)
            issue_in(ci + 2, bq)

    for k in (_NCHUNK - 3, _NCHUNK - 2, _NCHUNK - 1):
        wait_out(k % _NBUF)


def kernel(features, labels, epoch,
           running_mean_last_epoch, running_var_last_epoch,
           smoothed_mean_last_epoch, smoothed_var_last_epoch):
    lab = jnp.clip(labels.reshape(-1).astype(jnp.int32), 0, _NBUCKET - 1)
    ep = jnp.asarray(epoch, jnp.int32).reshape(1, 1)
    so = _make_tables(
        ep, running_var_last_epoch, smoothed_var_last_epoch,
        running_mean_last_epoch, smoothed_mean_last_epoch)
    out = _sc_apply(features.reshape(-1), lab, so.reshape(-1))
    return out.reshape(_BATCH, _FEAT)


# speedup vs baseline: 4.7255x; 2.8854x over previous
import functools

import jax
import jax.numpy as jnp
from jax import lax
from jax.experimental import pallas as pl
from jax.experimental.pallas import tpu as pltpu
from jax.experimental.pallas import tpu_sc as plsc

_FEAT = 512
_NBUCKET = 100
_BATCH = 16384
_START_SMOOTH = 1

_NC, _NS, _L = 2, 16, 16
_NW = _NC * _NS
_RPW = _BATCH // _NW
_CH = 8
_CW = _CH * _FEAT
_NCHUNK = _RPW // _CH
_NBUF = 2
_TABW = _NBUCKET * _FEAT


def _tables_body(ep_ref, rv_ref, sv_ref, rm_ref, sm_ref, so_ref):
    live = ep_ref[0, 0] >= _START_SMOOTH
    s = jnp.sqrt(jnp.clip(sv_ref[...] / rv_ref[...], 0.1, 10.0))
    s = jnp.where(live, s, 1.0)
    o = jnp.where(live, sm_ref[...] - rm_ref[...] * s, 0.0)
    so_ref[0, ...] = s
    so_ref[1, ...] = o


def _make_tables(ep, rv, sv, rm, sm):
    return pl.pallas_call(
        _tables_body,
        in_specs=[
            pl.BlockSpec(memory_space=pltpu.SMEM),
            pl.BlockSpec(memory_space=pltpu.VMEM),
            pl.BlockSpec(memory_space=pltpu.VMEM),
            pl.BlockSpec(memory_space=pltpu.VMEM),
            pl.BlockSpec(memory_space=pltpu.VMEM),
        ],
        out_shape=jax.ShapeDtypeStruct((2, _NBUCKET, _FEAT), jnp.float32),
    )(ep, rv, sv, rm, sm)


@functools.partial(
    pl.kernel,
    out_type=jax.ShapeDtypeStruct((_BATCH * _FEAT,), jnp.float32),
    mesh=plsc.VectorSubcoreMesh(core_axis_name="c", subcore_axis_name="s"),
    compiler_params=pltpu.CompilerParams(needs_layout_passes=False),
    scratch_types=[
        pltpu.VMEM((_RPW * _L,), jnp.int32),
        pltpu.VMEM((_TABW,), jnp.float32),
        pltpu.VMEM((_TABW,), jnp.float32),
        pltpu.VMEM((_NBUF * _CW,), jnp.float32),
        pltpu.VMEM((_NBUF * _CW,), jnp.float32),
        pltpu.SemaphoreType.DMA,
        pltpu.SemaphoreType.DMA,
        pltpu.SemaphoreType.DMA,
        pltpu.SemaphoreType.DMA,
        pltpu.SemaphoreType.DMA,
        pltpu.SemaphoreType.DMA,
    ],
)
def _sc_apply(feat_hbm, lab16_hbm, so_hbm, out_hbm,
              idx_v, tabs_v, tabo_v, f_v, r_v,
              sts, sto, si0, si1, so0, so1):
    sin = (si0, si1)
    sout = (so0, so1)
    wid = lax.axis_index("s") * _NC + lax.axis_index("c")
    base = wid * _RPW
    pltpu.async_copy(so_hbm.at[pl.ds(0, _TABW)], tabs_v, sts)
    pltpu.async_copy(so_hbm.at[pl.ds(_TABW, _TABW)], tabo_v, sto)
    pltpu.sync_copy(lab16_hbm.at[pl.ds(base * _L, _RPW * _L)], idx_v)

    def issue_in(ci, b):
        pltpu.async_copy(
            feat_hbm.at[pl.ds((base + ci * _CH) * _FEAT, _CW)],
            f_v.at[pl.ds(b * _CW, _CW)], sin[b])

    def wait_in(b):
        pltpu.make_async_copy(
            feat_hbm.at[pl.ds(0, _CW)],
            f_v.at[pl.ds(b * _CW, _CW)], sin[b]).wait()

    def issue_out(ci, b):
        pltpu.async_copy(
            r_v.at[pl.ds(b * _CW, _CW)],
            out_hbm.at[pl.ds((base + ci * _CH) * _FEAT, _CW)], sout[b])

    def wait_out(b):
        pltpu.make_async_copy(
            r_v.at[pl.ds(b * _CW, _CW)],
            out_hbm.at[pl.ds(0, _CW)], sout[b]).wait()

    for b in range(_NBUF):
        issue_in(b, b)
    pltpu.make_async_copy(so_hbm.at[pl.ds(0, _TABW)], tabs_v, sts).wait()
    pltpu.make_async_copy(so_hbm.at[pl.ds(0, _TABW)], tabo_v, sto).wait()

    iot = lax.iota(jnp.int32, 16)

    def outer(ci2, carry):
        for b in range(_NBUF):
            ci = ci2 * _NBUF + b
            wait_in(b)

            @pl.when(ci2 > 0)
            def _():
                wait_out(b)

            @plsc.parallel_loop(0, _CH)
            def _(r, ci=ci, b=b):
                lbv = idx_v[pl.ds((ci * _CH + r) * _L, _L)]
                sbase = lbv * jnp.int32(_FEAT) + iot
                rbase = b * _CW + r * _FEAT
                for j in range(_FEAT // _L):
                    sl = pl.ds(rbase + j * _L, _L)
                    sidx = sbase + jnp.int32(j * _L)
                    sc = plsc.load_gather(tabs_v, [sidx])
                    of = plsc.load_gather(tabo_v, [sidx])
                    r_v[sl] = f_v[sl] * sc + of

            issue_out(ci, b)

            @pl.when(ci + _NBUF < _NCHUNK)
            def _():
                issue_in(ci + _NBUF, b)
        return carry

    lax.fori_loop(0, _NCHUNK // _NBUF, outer, 0)
    for b in range(_NBUF):
        wait_out(b)


def kernel(features, labels, epoch,
           running_mean_last_epoch, running_var_last_epoch,
           smoothed_mean_last_epoch, smoothed_var_last_epoch):
    lab = jnp.clip(labels.reshape(-1).astype(jnp.int32), 0, _NBUCKET - 1)
    lab16 = jnp.repeat(lab, _L)
    ep = jnp.asarray(epoch, jnp.int32).reshape(1, 1)
    so = _make_tables(
        ep, running_var_last_epoch, smoothed_var_last_epoch,
        running_mean_last_epoch, smoothed_mean_last_epoch)
    out = _sc_apply(features.reshape(-1), lab16, so.reshape(-1))
    return out.reshape(_BATCH, _FEAT)
